# baseline (speedup 1.0000x reference)
import jax
import jax.numpy as jnp
from jax import lax
from jax.experimental import pallas as pl
from jax.experimental.pallas import tpu as pltpu
from jax.experimental.pallas import tpu_sc as plsc

NUM_CORES = 2
NUM_SUBCORES = 16
LANES = 16
NUM_WORKERS = NUM_CORES * NUM_SUBCORES

BATCH = 16384
FACTOR = 32
NROWS = 1000000
B_PER_W = BATCH // NUM_WORKERS
CHUNK = 128
NCHUNKS = B_PER_W // CHUNK
GROUPS = CHUNK // LANES

TBLK = 8192
TSEG = TBLK // 4
NTBLK = -(-NROWS // TBLK)
BIGROWS = NTBLK * TSEG


def _tr_body(x_ref, y_ref):
  eye = jnp.eye(FACTOR, dtype=jnp.float32)
  for q in range(4):
    xq = x_ref[:, pl.ds(q * TSEG, TSEG)]
    y_ref[:, pl.ds(q * FACTOR, FACTOR)] = lax.dot_general(
        xq, eye, (((0,), (0,)), ((), ())),
        precision=lax.Precision.HIGHEST)


def _retile(tab_t):
  return pl.pallas_call(
      _tr_body,
      grid=(NTBLK,),
      in_specs=[pl.BlockSpec((FACTOR, TBLK), lambda g: (0, g))],
      out_specs=pl.BlockSpec((TSEG, 128), lambda g: (g, 0)),
      out_shape=jax.ShapeDtypeStruct((BIGROWS, 128), jnp.float32),
  )(tab_t)


def _bpr_body(user_hbm, item_i_hbm, item_j_hbm, utab_hbm, itab_hbm,
              out_i_hbm, out_j_hbm,
              uidx_v, iidx_v, jidx_v, ubig_v, ibig_v, jbig_v,
              urows_v, irows_v, jrows_v,
              acc_i_v, acc_j_v, tbuf_i, tbuf_j, sem0, sem1, sem2):
  wid = lax.axis_index("s") * NUM_CORES + lax.axis_index("c")
  base = wid * B_PER_W

  c0 = pltpu.async_copy(user_hbm.at[pl.ds(base, B_PER_W)], uidx_v, sem0)
  c1 = pltpu.async_copy(item_i_hbm.at[pl.ds(base, B_PER_W)], iidx_v, sem1)
  c2 = pltpu.async_copy(item_j_hbm.at[pl.ds(base, B_PER_W)], jidx_v, sem2)
  c0.wait()
  c1.wait()
  c2.wait()

  lane_iota = lax.iota(jnp.int32, LANES)

  def bigrow(idx):
    return lax.shift_left(
        lax.shift_right_logical(idx, 13), 11) | (idx & 2047)

  for c in range(NCHUNKS):
    cbase = c * CHUNK
    for v in range(CHUNK // LANES):
      s = pl.ds(v * LANES, LANES)
      ubig_v[s] = bigrow(uidx_v[pl.ds(cbase + v * LANES, LANES)])
      ibig_v[s] = bigrow(iidx_v[pl.ds(cbase + v * LANES, LANES)])
      jbig_v[s] = bigrow(jidx_v[pl.ds(cbase + v * LANES, LANES)])

    g0 = pltpu.async_copy(utab_hbm.at[ubig_v], urows_v, sem0)
    g1 = pltpu.async_copy(itab_hbm.at[ibig_v], irows_v, sem1)
    g2 = pltpu.async_copy(itab_hbm.at[jbig_v], jrows_v, sem2)
    g0.wait()
    g1.wait()
    g2.wait()

    def group(g, carry):
      grow = g * LANES
      uo = (lax.shift_right_logical(
          uidx_v[pl.ds(cbase + grow, LANES)], 11) & 3) * FACTOR
      io = (lax.shift_right_logical(
          iidx_v[pl.ds(cbase + grow, LANES)], 11) & 3) * FACTOR
      jo = (lax.shift_right_logical(
          jidx_v[pl.ds(cbase + grow, LANES)], 11) & 3) * FACTOR
      for r in range(LANES):
        row = grow + r
        uoff = uo[r]
        ioff = io[r]
        joff = jo[r]
        u0 = urows_v[row, pl.ds(uoff, LANES)]
        u1 = urows_v[row, pl.ds(uoff + LANES, LANES)]
        i0 = irows_v[row, pl.ds(ioff, LANES)]
        i1 = irows_v[row, pl.ds(ioff + LANES, LANES)]
        j0 = jrows_v[row, pl.ds(joff, LANES)]
        j1 = jrows_v[row, pl.ds(joff + LANES, LANES)]
        p_i = u0 * i0 + u1 * i1
        p_j = u0 * j0 + u1 * j1
        tcol = lane_iota * LANES + r
        plsc.store_scatter(tbuf_i, [tcol], p_i)
        plsc.store_scatter(tbuf_j, [tcol], p_j)
      acc_i = tbuf_i[pl.ds(0, LANES)]
      acc_j = tbuf_j[pl.ds(0, LANES)]
      for k in range(1, LANES):
        acc_i = acc_i + tbuf_i[pl.ds(k * LANES, LANES)]
        acc_j = acc_j + tbuf_j[pl.ds(k * LANES, LANES)]
      acc_i_v[pl.ds(cbase + grow, LANES)] = acc_i
      acc_j_v[pl.ds(cbase + grow, LANES)] = acc_j
      return carry

    lax.fori_loop(0, GROUPS, group, 0)

  pltpu.sync_copy(acc_i_v, out_i_hbm.at[pl.ds(base, B_PER_W)])
  pltpu.sync_copy(acc_j_v, out_j_hbm.at[pl.ds(base, B_PER_W)])


@jax.jit
def _bpr(user, item_i, item_j, embed_user_weight, embed_item_weight):
  utab2 = _retile(embed_user_weight.T)
  itab2 = _retile(embed_item_weight.T)
  mesh = plsc.VectorSubcoreMesh(core_axis_name="c", subcore_axis_name="s")
  f = pl.kernel(
      _bpr_body,
      out_type=(
          jax.ShapeDtypeStruct((BATCH,), jnp.float32),
          jax.ShapeDtypeStruct((BATCH,), jnp.float32),
      ),
      mesh=mesh,
      scratch_types=[
          pltpu.VMEM((B_PER_W,), jnp.int32),
          pltpu.VMEM((B_PER_W,), jnp.int32),
          pltpu.VMEM((B_PER_W,), jnp.int32),
          pltpu.VMEM((CHUNK,), jnp.int32),
          pltpu.VMEM((CHUNK,), jnp.int32),
          pltpu.VMEM((CHUNK,), jnp.int32),
          pltpu.VMEM((CHUNK, 128), jnp.float32),
          pltpu.VMEM((CHUNK, 128), jnp.float32),
          pltpu.VMEM((CHUNK, 128), jnp.float32),
          pltpu.VMEM((B_PER_W,), jnp.float32),
          pltpu.VMEM((B_PER_W,), jnp.float32),
          pltpu.VMEM((LANES * LANES,), jnp.float32),
          pltpu.VMEM((LANES * LANES,), jnp.float32),
          pltpu.SemaphoreType.DMA,
          pltpu.SemaphoreType.DMA,
          pltpu.SemaphoreType.DMA,
      ],
      compiler_params=pltpu.CompilerParams(needs_layout_passes=False),
      name="bpr_sc",
  )
  return f(user, item_i, item_j, utab2, itab2)


def kernel(user, item_i, item_j, embed_user_weight, embed_item_weight):
  user = user.astype(jnp.int32)
  item_i = item_i.astype(jnp.int32)
  item_j = item_j.astype(jnp.int32)
  return _bpr(user, item_i, item_j, embed_user_weight, embed_item_weight)


# speedup vs baseline: 2.0686x; 2.0686x over previous
import jax
import jax.numpy as jnp
from jax import lax
from jax.experimental import pallas as pl
from jax.experimental.pallas import tpu as pltpu
from jax.experimental.pallas import tpu_sc as plsc

NUM_CORES = 2
NUM_SUBCORES = 16
LANES = 16
NUM_WORKERS = NUM_CORES * NUM_SUBCORES

BATCH = 16384
FACTOR = 32
NROWS = 1000000
B_PER_W = BATCH // NUM_WORKERS
CHUNK = 128
NCHUNKS = B_PER_W // CHUNK
GROUPS = CHUNK // LANES

SEG = 2048
HBLK = 4 * SEG
SUBBLKS = 4
TBLK = SUBBLKS * HBLK
NTBLK = -(-NROWS // TBLK)
BIGROWS = NTBLK * SUBBLKS * SEG


def _tr_body(x_ref, y_ref):
  for h in range(SUBBLKS):
    for q in range(4):
      y_ref[pl.ds(h * SEG, SEG), pl.ds(q * FACTOR, FACTOR)] = (
          x_ref[:, pl.ds(h * HBLK + q * SEG, SEG)].T)


def _retile(tab_t):
  return pl.pallas_call(
      _tr_body,
      grid=(NTBLK,),
      in_specs=[pl.BlockSpec((FACTOR, TBLK), lambda g: (0, g))],
      out_specs=pl.BlockSpec((SUBBLKS * SEG, 128), lambda g: (g, 0)),
      out_shape=jax.ShapeDtypeStruct((BIGROWS, 128), jnp.float32),
  )(tab_t)


def _bpr_body(user_hbm, item_i_hbm, item_j_hbm, utab_hbm, itab_hbm,
              out_i_hbm, out_j_hbm,
              uidx_v, iidx_v, jidx_v, ubig_v, ibig_v, jbig_v,
              urows_v, irows_v, jrows_v,
              acc_i_v, acc_j_v, tbuf_i, tbuf_j, sem0, sem1, sem2):
  wid = lax.axis_index("s") * NUM_CORES + lax.axis_index("c")
  base = wid * B_PER_W

  c0 = pltpu.async_copy(user_hbm.at[pl.ds(base, B_PER_W)], uidx_v, sem0)
  c1 = pltpu.async_copy(item_i_hbm.at[pl.ds(base, B_PER_W)], iidx_v, sem1)
  c2 = pltpu.async_copy(item_j_hbm.at[pl.ds(base, B_PER_W)], jidx_v, sem2)
  c0.wait()
  c1.wait()
  c2.wait()

  lane_iota = lax.iota(jnp.int32, LANES)

  def bigrow(idx):
    return lax.shift_left(
        lax.shift_right_logical(idx, 13), 11) | (idx & 2047)

  for c in range(NCHUNKS):
    cbase = c * CHUNK
    for v in range(CHUNK // LANES):
      s = pl.ds(v * LANES, LANES)
      ubig_v[s] = bigrow(uidx_v[pl.ds(cbase + v * LANES, LANES)])
      ibig_v[s] = bigrow(iidx_v[pl.ds(cbase + v * LANES, LANES)])
      jbig_v[s] = bigrow(jidx_v[pl.ds(cbase + v * LANES, LANES)])

    g0 = pltpu.async_copy(utab_hbm.at[ubig_v], urows_v, sem0)
    g1 = pltpu.async_copy(itab_hbm.at[ibig_v], irows_v, sem1)
    g2 = pltpu.async_copy(itab_hbm.at[jbig_v], jrows_v, sem2)
    g0.wait()
    g1.wait()
    g2.wait()

    def group(g, carry):
      grow = g * LANES
      uo = (lax.shift_right_logical(
          uidx_v[pl.ds(cbase + grow, LANES)], 11) & 3) * FACTOR
      io = (lax.shift_right_logical(
          iidx_v[pl.ds(cbase + grow, LANES)], 11) & 3) * FACTOR
      jo = (lax.shift_right_logical(
          jidx_v[pl.ds(cbase + grow, LANES)], 11) & 3) * FACTOR
      for r in range(LANES):
        row = grow + r
        uoff = uo[r]
        ioff = io[r]
        joff = jo[r]
        u0 = urows_v[row, pl.ds(uoff, LANES)]
        u1 = urows_v[row, pl.ds(uoff + LANES, LANES)]
        i0 = irows_v[row, pl.ds(ioff, LANES)]
        i1 = irows_v[row, pl.ds(ioff + LANES, LANES)]
        j0 = jrows_v[row, pl.ds(joff, LANES)]
        j1 = jrows_v[row, pl.ds(joff + LANES, LANES)]
        p_i = u0 * i0 + u1 * i1
        p_j = u0 * j0 + u1 * j1
        tcol = lane_iota * LANES + r
        plsc.store_scatter(tbuf_i, [tcol], p_i)
        plsc.store_scatter(tbuf_j, [tcol], p_j)
      acc_i = tbuf_i[pl.ds(0, LANES)]
      acc_j = tbuf_j[pl.ds(0, LANES)]
      for k in range(1, LANES):
        acc_i = acc_i + tbuf_i[pl.ds(k * LANES, LANES)]
        acc_j = acc_j + tbuf_j[pl.ds(k * LANES, LANES)]
      acc_i_v[pl.ds(cbase + grow, LANES)] = acc_i
      acc_j_v[pl.ds(cbase + grow, LANES)] = acc_j
      return carry

    lax.fori_loop(0, GROUPS, group, 0)

  pltpu.sync_copy(acc_i_v, out_i_hbm.at[pl.ds(base, B_PER_W)])
  pltpu.sync_copy(acc_j_v, out_j_hbm.at[pl.ds(base, B_PER_W)])


@jax.jit
def _bpr(user, item_i, item_j, embed_user_weight, embed_item_weight):
  utab2 = _retile(embed_user_weight.T)
  itab2 = _retile(embed_item_weight.T)
  mesh = plsc.VectorSubcoreMesh(core_axis_name="c", subcore_axis_name="s")
  f = pl.kernel(
      _bpr_body,
      out_type=(
          jax.ShapeDtypeStruct((BATCH,), jnp.float32),
          jax.ShapeDtypeStruct((BATCH,), jnp.float32),
      ),
      mesh=mesh,
      scratch_types=[
          pltpu.VMEM((B_PER_W,), jnp.int32),
          pltpu.VMEM((B_PER_W,), jnp.int32),
          pltpu.VMEM((B_PER_W,), jnp.int32),
          pltpu.VMEM((CHUNK,), jnp.int32),
          pltpu.VMEM((CHUNK,), jnp.int32),
          pltpu.VMEM((CHUNK,), jnp.int32),
          pltpu.VMEM((CHUNK, 128), jnp.float32),
          pltpu.VMEM((CHUNK, 128), jnp.float32),
          pltpu.VMEM((CHUNK, 128), jnp.float32),
          pltpu.VMEM((B_PER_W,), jnp.float32),
          pltpu.VMEM((B_PER_W,), jnp.float32),
          pltpu.VMEM((LANES * LANES,), jnp.float32),
          pltpu.VMEM((LANES * LANES,), jnp.float32),
          pltpu.SemaphoreType.DMA,
          pltpu.SemaphoreType.DMA,
          pltpu.SemaphoreType.DMA,
      ],
      compiler_params=pltpu.CompilerParams(needs_layout_passes=False),
      name="bpr_sc",
  )
  return f(user, item_i, item_j, utab2, itab2)


def kernel(user, item_i, item_j, embed_user_weight, embed_item_weight):
  user = user.astype(jnp.int32)
  item_i = item_i.astype(jnp.int32)
  item_j = item_j.astype(jnp.int32)
  return _bpr(user, item_i, item_j, embed_user_weight, embed_item_weight)


# speedup vs baseline: 2.0918x; 1.0112x over previous
import jax
import jax.numpy as jnp
from jax import lax
from jax.experimental import pallas as pl
from jax.experimental.pallas import tpu as pltpu
from jax.experimental.pallas import tpu_sc as plsc

NUM_CORES = 2
NUM_SUBCORES = 16
LANES = 16
NUM_WORKERS = NUM_CORES * NUM_SUBCORES

BATCH = 16384
FACTOR = 32
NROWS = 1000000
B_PER_W = BATCH // NUM_WORKERS
CHUNK = 128
NCHUNKS = B_PER_W // CHUNK
GROUPS = CHUNK // LANES

SEG = 2048
HBLK = 4 * SEG
SUBBLKS = 4
TBLK = SUBBLKS * HBLK
NTBLK = -(-NROWS // TBLK)
BIGROWS = NTBLK * SUBBLKS * SEG


def _tr_body(xu_ref, xi_ref, yu_ref, yi_ref):
  for h in range(SUBBLKS):
    for q in range(4):
      s = h * HBLK + q * SEG
      yu_ref[pl.ds(h * SEG, SEG), pl.ds(q * FACTOR, FACTOR)] = (
          xu_ref[:, pl.ds(s, SEG)].T)
      yi_ref[pl.ds(h * SEG, SEG), pl.ds(q * FACTOR, FACTOR)] = (
          xi_ref[:, pl.ds(s, SEG)].T)


def _retile(utab_t, itab_t):
  spec_in = pl.BlockSpec((FACTOR, TBLK), lambda g: (0, g))
  spec_out = pl.BlockSpec((SUBBLKS * SEG, 128), lambda g: (g, 0))
  return pl.pallas_call(
      _tr_body,
      grid=(NTBLK,),
      in_specs=[spec_in, spec_in],
      out_specs=[spec_out, spec_out],
      out_shape=[
          jax.ShapeDtypeStruct((BIGROWS, 128), jnp.float32),
          jax.ShapeDtypeStruct((BIGROWS, 128), jnp.float32),
      ],
  )(utab_t, itab_t)


def _bpr_body(user_hbm, item_i_hbm, item_j_hbm, utab_hbm, itab_hbm,
              out_i_hbm, out_j_hbm,
              uidx_v, iidx_v, jidx_v, ubig_v, ibig_v, jbig_v,
              urows_v, irows_v, jrows_v,
              acc_i_v, acc_j_v, tbuf_i, tbuf_j, sem0, sem1, sem2):
  wid = lax.axis_index("s") * NUM_CORES + lax.axis_index("c")
  base = wid * B_PER_W

  c0 = pltpu.async_copy(user_hbm.at[pl.ds(base, B_PER_W)], uidx_v, sem0)
  c1 = pltpu.async_copy(item_i_hbm.at[pl.ds(base, B_PER_W)], iidx_v, sem1)
  c2 = pltpu.async_copy(item_j_hbm.at[pl.ds(base, B_PER_W)], jidx_v, sem2)
  c0.wait()
  c1.wait()
  c2.wait()

  lane_iota = lax.iota(jnp.int32, LANES)

  def bigrow(idx):
    return lax.shift_left(
        lax.shift_right_logical(idx, 13), 11) | (idx & 2047)

  for c in range(NCHUNKS):
    cbase = c * CHUNK
    for v in range(CHUNK // LANES):
      s = pl.ds(v * LANES, LANES)
      ubig_v[s] = bigrow(uidx_v[pl.ds(cbase + v * LANES, LANES)])
      ibig_v[s] = bigrow(iidx_v[pl.ds(cbase + v * LANES, LANES)])
      jbig_v[s] = bigrow(jidx_v[pl.ds(cbase + v * LANES, LANES)])

    g0 = pltpu.async_copy(utab_hbm.at[ubig_v], urows_v, sem0)
    g1 = pltpu.async_copy(itab_hbm.at[ibig_v], irows_v, sem1)
    g2 = pltpu.async_copy(itab_hbm.at[jbig_v], jrows_v, sem2)
    g0.wait()
    g1.wait()
    g2.wait()

    def group(g, carry):
      grow = g * LANES
      uo = (lax.shift_right_logical(
          uidx_v[pl.ds(cbase + grow, LANES)], 11) & 3) * FACTOR
      io = (lax.shift_right_logical(
          iidx_v[pl.ds(cbase + grow, LANES)], 11) & 3) * FACTOR
      jo = (lax.shift_right_logical(
          jidx_v[pl.ds(cbase + grow, LANES)], 11) & 3) * FACTOR
      for r in range(LANES):
        row = grow + r
        uoff = uo[r]
        ioff = io[r]
        joff = jo[r]
        u0 = urows_v[row, pl.ds(uoff, LANES)]
        u1 = urows_v[row, pl.ds(uoff + LANES, LANES)]
        i0 = irows_v[row, pl.ds(ioff, LANES)]
        i1 = irows_v[row, pl.ds(ioff + LANES, LANES)]
        j0 = jrows_v[row, pl.ds(joff, LANES)]
        j1 = jrows_v[row, pl.ds(joff + LANES, LANES)]
        p_i = u0 * i0 + u1 * i1
        p_j = u0 * j0 + u1 * j1
        tcol = lane_iota * LANES + r
        plsc.store_scatter(tbuf_i, [tcol], p_i)
        plsc.store_scatter(tbuf_j, [tcol], p_j)
      acc_i = tbuf_i[pl.ds(0, LANES)]
      acc_j = tbuf_j[pl.ds(0, LANES)]
      for k in range(1, LANES):
        acc_i = acc_i + tbuf_i[pl.ds(k * LANES, LANES)]
        acc_j = acc_j + tbuf_j[pl.ds(k * LANES, LANES)]
      acc_i_v[pl.ds(cbase + grow, LANES)] = acc_i
      acc_j_v[pl.ds(cbase + grow, LANES)] = acc_j
      return carry

    lax.fori_loop(0, GROUPS, group, 0)

  pltpu.sync_copy(acc_i_v, out_i_hbm.at[pl.ds(base, B_PER_W)])
  pltpu.sync_copy(acc_j_v, out_j_hbm.at[pl.ds(base, B_PER_W)])


@jax.jit
def _bpr(user, item_i, item_j, embed_user_weight, embed_item_weight):
  utab2, itab2 = _retile(embed_user_weight.T, embed_item_weight.T)
  mesh = plsc.VectorSubcoreMesh(core_axis_name="c", subcore_axis_name="s")
  f = pl.kernel(
      _bpr_body,
      out_type=(
          jax.ShapeDtypeStruct((BATCH,), jnp.float32),
          jax.ShapeDtypeStruct((BATCH,), jnp.float32),
      ),
      mesh=mesh,
      scratch_types=[
          pltpu.VMEM((B_PER_W,), jnp.int32),
          pltpu.VMEM((B_PER_W,), jnp.int32),
          pltpu.VMEM((B_PER_W,), jnp.int32),
          pltpu.VMEM((CHUNK,), jnp.int32),
          pltpu.VMEM((CHUNK,), jnp.int32),
          pltpu.VMEM((CHUNK,), jnp.int32),
          pltpu.VMEM((CHUNK, 128), jnp.float32),
          pltpu.VMEM((CHUNK, 128), jnp.float32),
          pltpu.VMEM((CHUNK, 128), jnp.float32),
          pltpu.VMEM((B_PER_W,), jnp.float32),
          pltpu.VMEM((B_PER_W,), jnp.float32),
          pltpu.VMEM((LANES * LANES,), jnp.float32),
          pltpu.VMEM((LANES * LANES,), jnp.float32),
          pltpu.SemaphoreType.DMA,
          pltpu.SemaphoreType.DMA,
          pltpu.SemaphoreType.DMA,
      ],
      compiler_params=pltpu.CompilerParams(needs_layout_passes=False),
      name="bpr_sc",
  )
  return f(user, item_i, item_j, utab2, itab2)


def kernel(user, item_i, item_j, embed_user_weight, embed_item_weight):
  user = user.astype(jnp.int32)
  item_i = item_i.astype(jnp.int32)
  item_j = item_j.astype(jnp.int32)
  return _bpr(user, item_i, item_j, embed_user_weight, embed_item_weight)
